# baseline (speedup 1.0000x reference)
import functools

import jax
import jax.numpy as jnp
from jax import lax
from jax.experimental import pallas as pl
from jax.experimental.pallas import tpu as pltpu
from jax.experimental.pallas import tpu_sc as plsc

D = 64
HIST = 50
L = 16
NC, NS = 2, 16
NW = NC * NS
GW = 80
CB = 8
TOK = CB * HIST
GROWS = TOK // GW
PW = 16384
PB = PW.bit_length() - 1


def _pack_table(emb_t):
    V = emb_t.shape[1]
    W = PW
    grid = pl.cdiv(V, W)

    def body(x_ref, o_ref):
        o_ref[:, 0:D] = x_ref[:, 0 : W // 2].T
        o_ref[:, D : 2 * D] = x_ref[:, W // 2 : W].T

    return pl.pallas_call(
        body,
        grid=(grid,),
        in_specs=[pl.BlockSpec((D, W), lambda i: (0, i))],
        out_specs=pl.BlockSpec((W // 2, 2 * D), lambda i: (i, 0)),
        out_shape=jax.ShapeDtypeStruct((grid * W // 2, 2 * D), jnp.float32),
    )(emb_t)


def _bag_means(text, emb2, batch):
    bags_w = batch // NW
    toks_w = bags_w * HIST
    chunks = bags_w // CB
    mesh = plsc.VectorSubcoreMesh(core_axis_name="c", subcore_axis_name="s")

    @functools.partial(
        pl.kernel,
        mesh=mesh,
        out_type=jax.ShapeDtypeStruct((batch, D), jnp.float32),
        scratch_types=[
            pltpu.VMEM((toks_w,), jnp.int32),
            pltpu.VMEM((2 * TOK,), jnp.int32),
            pltpu.VMEM((2 * TOK, 2 * D), jnp.float32),
            pltpu.VMEM((CB, D), jnp.float32),
            pltpu.SemaphoreType.DMA((2,)),
        ],
    )
    def k(text_hbm, emb_hbm, out_hbm, idx_v, row_v, rows_v, obuf, gsem):
        wid = lax.axis_index("s") * NC + lax.axis_index("c")
        bag0 = wid * bags_w
        pltpu.sync_copy(text_hbm.at[pl.ds(bag0 * HIST, toks_w)], idx_v)

        def fire(g, buf):
            t00 = g * TOK
            for j in range(TOK // L):
                v = idx_v[pl.ds(t00 + j * L, L)]
                row_v[pl.ds(buf * TOK + j * L, L)] = lax.shift_left(
                    lax.shift_right_logical(v, PB), PB - 1
                ) | (v & (PW // 2 - 1))
            for j in range(GROWS):
                pltpu.async_copy(
                    emb_hbm.at[row_v.at[pl.ds(buf * TOK + j * GW, GW)]],
                    rows_v.at[pl.ds(buf * TOK + j * GW, GW)],
                    gsem.at[buf],
                )

        def drain(buf):
            for j in range(GROWS):
                pltpu.make_async_copy(
                    emb_hbm.at[row_v.at[pl.ds(buf * TOK + j * GW, GW)]],
                    rows_v.at[pl.ds(buf * TOK + j * GW, GW)],
                    gsem.at[buf],
                ).wait()

        fire(0, 0)
        inv = jnp.full((L,), 1.0 / HIST, jnp.float32)

        def chunk_body(g, carry):
            t00 = g * TOK
            buf = lax.rem(g, 2)

            @pl.when(g + 1 < chunks)
            def _():
                fire(g + 1, 1 - buf)

            drain(buf)
            for bb in range(CB):
                t0 = bb * HIST

                def rbody(r, acc):
                    out = list(acc)
                    tb = t0 + r * 5
                    pvec = idx_v[pl.ds(t00 + tb, L)]
                    for u in range(5):
                        t = tb + u
                        off = (
                            lax.shift_right_logical(pvec[u], PB - 1) & 1
                        ) * D
                        for c in range(4):
                            out[c] = out[c] + rows_v[
                                buf * TOK + t, pl.ds(off + c * L, L)
                            ]
                    return tuple(out)

                z = jnp.zeros((L,), jnp.float32)
                a = lax.fori_loop(0, HIST // 5, rbody, (z, z, z, z))
                for c in range(4):
                    obuf[bb, pl.ds(c * L, L)] = a[c] * inv

            pltpu.sync_copy(obuf, out_hbm.at[pl.ds(bag0 + g * CB, CB)])
            return carry

        lax.fori_loop(0, chunks, chunk_body, 0)

    return k(text, emb2)


def _mlp(x, w1t, b1, w3t, b3):
    batch = x.shape[0]
    blk = 2048
    h1 = w1t.shape[1]
    h3 = w3t.shape[1]

    def body(x_ref, w1_ref, b1_ref, w3_ref, b3_ref, o_ref):
        h = jnp.dot(x_ref[...], w1_ref[...], preferred_element_type=jnp.float32)
        h = jax.nn.sigmoid(h + b1_ref[...])
        o = jnp.dot(h, w3_ref[...], preferred_element_type=jnp.float32)
        o_ref[...] = jax.nn.sigmoid(o + b3_ref[...])

    return pl.pallas_call(
        body,
        grid=(batch // blk,),
        in_specs=[
            pl.BlockSpec((blk, D), lambda i: (i, 0)),
            pl.BlockSpec((D, h1), lambda i: (0, 0)),
            pl.BlockSpec((1, h1), lambda i: (0, 0)),
            pl.BlockSpec((h1, h3), lambda i: (0, 0)),
            pl.BlockSpec((1, h3), lambda i: (0, 0)),
        ],
        out_specs=pl.BlockSpec((blk, h3), lambda i: (i, 0)),
        out_shape=jax.ShapeDtypeStruct((batch, h3), jnp.float32),
    )(x, w1t, b1.reshape(1, h1), w3t, b3.reshape(1, h3))


def kernel(text, offsets, emb_w, fc1_w, fc1_b, fc3_w, fc3_b):
    del offsets
    n = text.shape[0]
    batch = n // HIST
    emb2 = _pack_table(emb_w.T)
    means = _bag_means(text.astype(jnp.int32), emb2, batch)
    return _mlp(means, fc1_w.T, fc1_b, fc3_w.T, fc3_b)


# speedup vs baseline: 1.0318x; 1.0318x over previous
import functools

import jax
import jax.numpy as jnp
from jax import lax
from jax.experimental import pallas as pl
from jax.experimental.pallas import tpu as pltpu
from jax.experimental.pallas import tpu_sc as plsc

D = 64
HIST = 50
L = 16
NC, NS = 2, 16
NW = NC * NS
GW = 80
CB = 8
TOK = CB * HIST
GROWS = TOK // GW
PW = 32768
PB = PW.bit_length() - 1


def _pack_table(emb_t):
    V = emb_t.shape[1]
    W = PW
    grid = pl.cdiv(V, W)

    def body(x_ref, o_ref):
        o_ref[:, 0:D] = x_ref[:, 0 : W // 2].T
        o_ref[:, D : 2 * D] = x_ref[:, W // 2 : W].T

    return pl.pallas_call(
        body,
        grid=(grid,),
        in_specs=[pl.BlockSpec((D, W), lambda i: (0, i))],
        out_specs=pl.BlockSpec((W // 2, 2 * D), lambda i: (i, 0)),
        out_shape=jax.ShapeDtypeStruct((grid * W // 2, 2 * D), jnp.float32),
    )(emb_t)


def _bag_means(text, emb2, batch):
    bags_w = batch // NW
    toks_w = bags_w * HIST
    chunks = bags_w // CB
    mesh = plsc.VectorSubcoreMesh(core_axis_name="c", subcore_axis_name="s")

    @functools.partial(
        pl.kernel,
        mesh=mesh,
        out_type=jax.ShapeDtypeStruct((batch, D), jnp.float32),
        scratch_types=[
            pltpu.VMEM((toks_w,), jnp.int32),
            pltpu.VMEM((2 * TOK,), jnp.int32),
            pltpu.VMEM((2 * TOK, 2 * D), jnp.float32),
            pltpu.VMEM((CB, D), jnp.float32),
            pltpu.SemaphoreType.DMA((2,)),
        ],
    )
    def k(text_hbm, emb_hbm, out_hbm, idx_v, row_v, rows_v, obuf, gsem):
        wid = lax.axis_index("s") * NC + lax.axis_index("c")
        bag0 = wid * bags_w
        pltpu.sync_copy(text_hbm.at[pl.ds(bag0 * HIST, toks_w)], idx_v)

        def fire(g, buf):
            t00 = g * TOK
            for j in range(TOK // L):
                v = idx_v[pl.ds(t00 + j * L, L)]
                row_v[pl.ds(buf * TOK + j * L, L)] = lax.shift_left(
                    lax.shift_right_logical(v, PB), PB - 1
                ) | (v & (PW // 2 - 1))
            for j in range(GROWS):
                pltpu.async_copy(
                    emb_hbm.at[row_v.at[pl.ds(buf * TOK + j * GW, GW)]],
                    rows_v.at[pl.ds(buf * TOK + j * GW, GW)],
                    gsem.at[buf],
                )

        def drain(buf):
            for j in range(GROWS):
                pltpu.make_async_copy(
                    emb_hbm.at[row_v.at[pl.ds(buf * TOK + j * GW, GW)]],
                    rows_v.at[pl.ds(buf * TOK + j * GW, GW)],
                    gsem.at[buf],
                ).wait()

        fire(0, 0)
        inv = jnp.full((L,), 1.0 / HIST, jnp.float32)

        def chunk_body(g, carry):
            t00 = g * TOK
            buf = lax.rem(g, 2)

            @pl.when(g + 1 < chunks)
            def _():
                fire(g + 1, 1 - buf)

            drain(buf)
            for bb in range(CB):
                t0 = bb * HIST

                def rbody(r, acc):
                    out = list(acc)
                    tb = t0 + r * 10
                    pvec = idx_v[pl.ds(t00 + tb, L)]
                    for u in range(10):
                        t = tb + u
                        off = (
                            lax.shift_right_logical(pvec[u], PB - 1) & 1
                        ) * D
                        for c in range(4):
                            out[c] = out[c] + rows_v[
                                buf * TOK + t, pl.ds(off + c * L, L)
                            ]
                    return tuple(out)

                z = jnp.zeros((L,), jnp.float32)
                a = lax.fori_loop(0, HIST // 10, rbody, (z, z, z, z))
                for c in range(4):
                    obuf[bb, pl.ds(c * L, L)] = a[c] * inv

            pltpu.sync_copy(obuf, out_hbm.at[pl.ds(bag0 + g * CB, CB)])
            return carry

        lax.fori_loop(0, chunks, chunk_body, 0)

    return k(text, emb2)


def _mlp(x, w1t, b1, w3t, b3):
    batch = x.shape[0]
    blk = 2048
    h1 = w1t.shape[1]
    h3 = w3t.shape[1]

    def body(x_ref, w1_ref, b1_ref, w3_ref, b3_ref, o_ref):
        h = jnp.dot(x_ref[...], w1_ref[...], preferred_element_type=jnp.float32)
        h = jax.nn.sigmoid(h + b1_ref[...])
        o = jnp.dot(h, w3_ref[...], preferred_element_type=jnp.float32)
        o_ref[...] = jax.nn.sigmoid(o + b3_ref[...])

    return pl.pallas_call(
        body,
        grid=(batch // blk,),
        in_specs=[
            pl.BlockSpec((blk, D), lambda i: (i, 0)),
            pl.BlockSpec((D, h1), lambda i: (0, 0)),
            pl.BlockSpec((1, h1), lambda i: (0, 0)),
            pl.BlockSpec((h1, h3), lambda i: (0, 0)),
            pl.BlockSpec((1, h3), lambda i: (0, 0)),
        ],
        out_specs=pl.BlockSpec((blk, h3), lambda i: (i, 0)),
        out_shape=jax.ShapeDtypeStruct((batch, h3), jnp.float32),
    )(x, w1t, b1.reshape(1, h1), w3t, b3.reshape(1, h3))


def kernel(text, offsets, emb_w, fc1_w, fc1_b, fc3_w, fc3_b):
    del offsets
    n = text.shape[0]
    batch = n // HIST
    emb2 = _pack_table(emb_w.T)
    means = _bag_means(text.astype(jnp.int32), emb2, batch)
    return _mlp(means, fc1_w.T, fc1_b, fc3_w.T, fc3_b)
